# baseline (speedup 1.0000x reference)
import functools

import jax
import jax.numpy as jnp
from jax import lax
from jax.experimental import pallas as pl
from jax.experimental.pallas import tpu as pltpu
from jax.experimental.pallas import tpu_sc as plsc

_EPS = 1e-5

_NUM_HW = 100
_EMBED_DIM = 64
_BATCH = 16384

_info = plsc.get_sparse_core_info()
_NC, _NS = _info.num_cores, _info.num_subcores
_NW = _NC * _NS
_B_PER_W = _BATCH // _NW
_LANES = 16
_NFULL = _NUM_HW // _LANES
_OVER = _NUM_HW - _LANES
_NG = _NFULL + 1
_NCHUNK = 4
_B_CHUNK = _B_PER_W // _NCHUNK
_G_CHUNK = _B_CHUNK // _LANES

_mesh = plsc.VectorSubcoreMesh(core_axis_name="c", subcore_axis_name="s")


def _group_off(gi):
    return _LANES * gi if gi < _NFULL else _OVER


@functools.partial(
    pl.kernel,
    mesh=_mesh,
    out_type=jax.ShapeDtypeStruct((_EMBED_DIM, _BATCH), jnp.float32),
    scratch_types=[
        pltpu.VMEM((_EMBED_DIM, _NUM_HW), jnp.float32),
        pltpu.VMEM((_EMBED_DIM,), jnp.float32),
        pltpu.VMEM((_EMBED_DIM,), jnp.float32),
        pltpu.VMEM((_B_PER_W,), jnp.int32),
        pltpu.VMEM((_EMBED_DIM, _B_PER_W), jnp.float32),
        pltpu.SemaphoreType.DMA,
        pltpu.SemaphoreType.DMA,
        pltpu.SemaphoreType.DMA,
    ],
    compiler_params=pltpu.CompilerParams(
        use_tc_tiling_on_sc=True, needs_layout_passes=False),
)
def _sc_fused(idx_hbm, tablet_hbm, gamma_hbm, beta_hbm, out_hbm,
              tbl_t, g_v, b_v, idx_v, rows_v, sem, gsem, osem):
    sid = lax.axis_index("s")
    wid = sid * _NC + lax.axis_index("c")
    base = wid * _B_PER_W

    with jax.named_scope("stage"):
        cp_idx = pltpu.async_copy(idx_hbm.at[pl.ds(base, _B_PER_W)], idx_v, sem)
        cp_g = pltpu.async_copy(gamma_hbm, g_v, gsem)
        cp_b = pltpu.async_copy(beta_hbm, b_v, gsem)
        cp_tbl = pltpu.async_copy(tablet_hbm, tbl_t, sem)
        cp_tbl.wait()

    ln_scope = jax.named_scope("normalize")
    ln_scope.__enter__()
    half = jnp.float32(0.5)
    threehalf = jnp.float32(1.5)
    inv_d = jnp.float32(1.0 / _EMBED_DIM)
    zeros = jnp.zeros((_LANES,), jnp.float32)

    @plsc.parallel_loop(0, _EMBED_DIM, unroll=4, carry=tuple([zeros] * (2 * _NG)))
    def stats(d, acc):
        out = []
        for gi in range(_NG):
            v = tbl_t[d, pl.ds(_group_off(gi), _LANES)]
            out.append(acc[gi] + v)
            out.append(acc[_NG + gi] + v * v)
        return tuple(out[0::2] + out[1::2])

    means = [stats[gi] * inv_d for gi in range(_NG)]
    rstds = []
    for gi in range(_NG):
        var = stats[_NG + gi] * inv_d - means[gi] * means[gi]
        v = var + jnp.float32(_EPS)
        i = plsc.bitcast(v, jnp.int32)
        i = jnp.int32(0x5F3759DF) - (i >> 1)
        y = plsc.bitcast(i, jnp.float32)
        for _ in range(3):
            y = y * (threehalf - half * v * y * y)
        rstds.append(y)

    tail_idx = lax.iota(jnp.int32, _LANES)
    tail_mask = tail_idx >= jnp.int32(_LANES - (_NUM_HW - _NFULL * _LANES))

    cp_g.wait()
    cp_b.wait()

    @plsc.parallel_loop(0, _EMBED_DIM, unroll=4)
    def _(d):
        d16 = jnp.full((_LANES,), d, jnp.int32)
        gd = plsc.load_gather(g_v, [d16])
        bd = plsc.load_gather(b_v, [d16])
        for gi in range(_NFULL):
            x = tbl_t[d, pl.ds(_LANES * gi, _LANES)]
            tbl_t[d, pl.ds(_LANES * gi, _LANES)] = (
                (x - means[gi]) * rstds[gi] * gd + bd)
        x = tbl_t[d, pl.ds(_OVER, _LANES)]
        y = (x - means[_NFULL]) * rstds[_NFULL] * gd + bd
        plsc.store_scatter(tbl_t, [d16, tail_idx + _OVER], y, mask=tail_mask)

    ln_scope.__exit__(None, None, None)

    cp_idx.wait()

    copies = []
    for c in range(_NCHUNK):
        with jax.named_scope("expand"):
            @plsc.parallel_loop(c * _G_CHUNK, (c + 1) * _G_CHUNK, unroll=2)
            def _(bg):
                bo = bg * _LANES
                idx16 = idx_v[pl.ds(bo, _LANES)]

                @plsc.parallel_loop(0, _EMBED_DIM, unroll=16)
                def _(d):
                    rows_v[d, pl.ds(bo, _LANES)] = plsc.load_gather(
                        tbl_t.at[d], [idx16])

        with jax.named_scope("flush"):
            copies.append(pltpu.async_copy(
                rows_v.at[:, pl.ds(c * _B_CHUNK, _B_CHUNK)],
                out_hbm.at[:, pl.ds(base + c * _B_CHUNK, _B_CHUNK)],
                osem))
    with jax.named_scope("drain"):
        for cp in copies:
            cp.wait()


def kernel(hw_indices, table, gamma, beta):
    out_t = _sc_fused(hw_indices.astype(jnp.int32), table.T, gamma, beta)
    return out_t.T


# speedup vs baseline: 1.0165x; 1.0165x over previous
import functools

import jax
import jax.numpy as jnp
from jax import lax
from jax.experimental import pallas as pl
from jax.experimental.pallas import tpu as pltpu
from jax.experimental.pallas import tpu_sc as plsc

_EPS = 1e-5

_NUM_HW = 100
_EMBED_DIM = 64
_BATCH = 16384

_info = plsc.get_sparse_core_info()
_NC, _NS = _info.num_cores, _info.num_subcores
_NW = _NC * _NS
_B_PER_W = _BATCH // _NW
_LANES = 16
_NFULL = _NUM_HW // _LANES
_OVER = _NUM_HW - _LANES
_NG = _NFULL + 1
_NCHUNK = 4
_B_CHUNK = _B_PER_W // _NCHUNK
_G_CHUNK = _B_CHUNK // _LANES

_mesh = plsc.VectorSubcoreMesh(core_axis_name="c", subcore_axis_name="s")


def _group_off(gi):
    return _LANES * gi if gi < _NFULL else _OVER


@functools.partial(
    pl.kernel,
    mesh=_mesh,
    out_type=jax.ShapeDtypeStruct((_EMBED_DIM, _BATCH), jnp.float32),
    scratch_types=[
        pltpu.VMEM((_EMBED_DIM, _NUM_HW), jnp.float32),
        pltpu.VMEM((_EMBED_DIM,), jnp.float32),
        pltpu.VMEM((_EMBED_DIM,), jnp.float32),
        pltpu.VMEM((_B_PER_W,), jnp.int32),
        pltpu.VMEM((_EMBED_DIM, _B_PER_W), jnp.float32),
        pltpu.SemaphoreType.DMA,
        pltpu.SemaphoreType.DMA,
        pltpu.SemaphoreType.DMA,
    ],
    compiler_params=pltpu.CompilerParams(
        use_tc_tiling_on_sc=True, needs_layout_passes=False),
)
def _sc_fused(idx_hbm, tablet_hbm, gamma_hbm, beta_hbm, out_hbm,
              tbl_t, g_v, b_v, idx_v, rows_v, sem, gsem, osem):
    sid = lax.axis_index("s")
    wid = sid * _NC + lax.axis_index("c")
    base = wid * _B_PER_W

    cp_idx = pltpu.async_copy(idx_hbm.at[pl.ds(base, _B_PER_W)], idx_v, sem)
    cp_g = pltpu.async_copy(gamma_hbm, g_v, gsem)
    cp_b = pltpu.async_copy(beta_hbm, b_v, gsem)
    cp_tbl = pltpu.async_copy(tablet_hbm, tbl_t, sem)
    cp_tbl.wait()

    half = jnp.float32(0.5)
    threehalf = jnp.float32(1.5)
    inv_d = jnp.float32(1.0 / _EMBED_DIM)
    zeros = jnp.zeros((_LANES,), jnp.float32)

    @plsc.parallel_loop(0, _EMBED_DIM, carry=tuple([zeros] * (2 * _NG)))
    def stats(d, acc):
        out = []
        for gi in range(_NG):
            v = tbl_t[d, pl.ds(_group_off(gi), _LANES)]
            out.append(acc[gi] + v)
            out.append(acc[_NG + gi] + v * v)
        return tuple(out[0::2] + out[1::2])

    means = [stats[gi] * inv_d for gi in range(_NG)]
    rstds = []
    for gi in range(_NG):
        var = stats[_NG + gi] * inv_d - means[gi] * means[gi]
        v = var + jnp.float32(_EPS)
        i = plsc.bitcast(v, jnp.int32)
        i = jnp.int32(0x5F3759DF) - (i >> 1)
        y = plsc.bitcast(i, jnp.float32)
        for _ in range(3):
            y = y * (threehalf - half * v * y * y)
        rstds.append(y)

    tail_idx = lax.iota(jnp.int32, _LANES)
    tail_mask = tail_idx >= jnp.int32(_LANES - (_NUM_HW - _NFULL * _LANES))

    cp_g.wait()
    cp_b.wait()

    @plsc.parallel_loop(0, _EMBED_DIM)
    def _(d):
        d16 = jnp.full((_LANES,), d, jnp.int32)
        gd = plsc.load_gather(g_v, [d16])
        bd = plsc.load_gather(b_v, [d16])
        for gi in range(_NFULL):
            x = tbl_t[d, pl.ds(_LANES * gi, _LANES)]
            tbl_t[d, pl.ds(_LANES * gi, _LANES)] = (
                (x - means[gi]) * rstds[gi] * gd + bd)
        x = tbl_t[d, pl.ds(_OVER, _LANES)]
        y = (x - means[_NFULL]) * rstds[_NFULL] * gd + bd
        plsc.store_scatter(tbl_t, [d16, tail_idx + _OVER], y, mask=tail_mask)

    cp_idx.wait()

    copies = []
    for c in range(_NCHUNK):
        @plsc.parallel_loop(c * _G_CHUNK, (c + 1) * _G_CHUNK)
        def _(bg):
            bo = bg * _LANES
            idx16 = idx_v[pl.ds(bo, _LANES)]

            @plsc.parallel_loop(0, _EMBED_DIM, unroll=16)
            def _(d):
                rows_v[d, pl.ds(bo, _LANES)] = plsc.load_gather(
                    tbl_t.at[d], [idx16])

        copies.append(pltpu.async_copy(
            rows_v.at[:, pl.ds(c * _B_CHUNK, _B_CHUNK)],
            out_hbm.at[:, pl.ds(base + c * _B_CHUNK, _B_CHUNK)],
            osem))
    for cp in copies:
        cp.wait()


def kernel(hw_indices, table, gamma, beta):
    out_t = _sc_fused(hw_indices.astype(jnp.int32), table.T, gamma, beta)
    return out_t.T
